# baseline (speedup 1.0000x reference)
import functools

import jax
import jax.numpy as jnp
from jax import lax
from jax.experimental import pallas as pl
from jax.experimental.pallas import tpu as pltpu
from jax.experimental.pallas import tpu_sc as plsc

N_NODES = 10000
D = 128
N_EDGES = 320000
ACE_IDX = 1234
H2_IDX = 5678

NC = 2
NS = 16
NW = NC * NS
CH = 48
NCHUNK = 216
NG = NCHUNK // 3
NSEG = 3
GPSEG = NG // NSEG
NIR = 81
EPW = NCHUNK * CH
E_PAD = EPW * NW
NPAD = 10240
ANPAD = 10112
ARPT = ANPAD // NS
MBLK = 1024
NBLK = NPAD // MBLK

_mesh = plsc.VectorSubcoreMesh(core_axis_name="c", subcore_axis_name="s")
_sc_params = pltpu.CompilerParams(needs_layout_passes=False)


@functools.partial(
    pl.kernel,
    out_type=jax.ShapeDtypeStruct((NW, NPAD), jnp.float32),
    mesh=_mesh,
    compiler_params=_sc_params,
    scratch_types=[
        pltpu.VMEM((NIR, 128), jnp.int32),
        pltpu.VMEM((NIR, 128), jnp.float32),
        pltpu.VMEM((NPAD,), jnp.float32),
    ],
)
def _deg_kernel(col_hbm, ew_hbm, out_hbm, col_v, ew_v, deg_v):
    c = lax.axis_index("c")
    s = lax.axis_index("s")
    wid = s * NC + c

    pltpu.sync_copy(col_hbm.at[wid], col_v)
    pltpu.sync_copy(ew_hbm.at[wid], ew_v)

    zero = jnp.zeros((16,), jnp.float32)

    def _zero(i, _):
        deg_v[pl.ds(pl.multiple_of(i * 16, 16), 16)] = zero
        return 0

    lax.fori_loop(0, NPAD // 16, _zero, 0)

    def _chunk(j, _):
        for i in range(128 // 16):
            sl = pl.ds(i * 16, 16)
            plsc.addupdate_scatter(deg_v, [col_v[j, sl]], ew_v[j, sl])
        return 0

    lax.fori_loop(0, NIR, _chunk, 0)
    pltpu.sync_copy(deg_v, out_hbm.at[wid])


@functools.partial(
    pl.kernel,
    out_type=(jax.ShapeDtypeStruct((NW, NIR, 128), jnp.float32),
              jax.ShapeDtypeStruct((NW, NIR, 128), jnp.float32),
              jax.ShapeDtypeStruct((NC, 128, 128), jnp.float32)),
    mesh=_mesh,
    compiler_params=_sc_params,
    scratch_types=[
        pltpu.VMEM((NIR, 128), jnp.int32),
        pltpu.VMEM((NIR, 128), jnp.int32),
        pltpu.VMEM((NIR, 128), jnp.float32),
        pltpu.VMEM((NIR, 128), jnp.float32),
        pltpu.VMEM((NIR, 128), jnp.float32),
        pltpu.VMEM((NPAD,), jnp.float32),
        pltpu.VMEM((128, 128), jnp.float32),
        pltpu.VMEM((1, 128), jnp.int32),
        pltpu.VMEM_SHARED((128, 128), jnp.float32),
    ],
)
def _norm_kernel(row_hbm, col_hbm, ew_hbm, dinv_hbm, out_hbm, out3_hbm,
                 outm_hbm, row_v, col_v, ew_v, norm_v, norm3_v, dinv_v,
                 mark_v, irow_v, macc):
    c = lax.axis_index("c")
    s = lax.axis_index("s")
    wid = s * NC + c

    pltpu.sync_copy(row_hbm.at[wid], row_v)
    pltpu.sync_copy(col_hbm.at[wid], col_v)
    pltpu.sync_copy(ew_hbm.at[wid], ew_v)
    pltpu.sync_copy(dinv_hbm, dinv_v)

    zero = jnp.zeros((16,), jnp.float32)
    ione = jnp.ones((16,), jnp.float32)
    base16 = jnp.arange(16, dtype=jnp.int32)

    def _zmark(r, _):
        for i in range(128 // 16):
            mark_v[r, pl.ds(i * 16, 16)] = zero
        return 0

    lax.fori_loop(0, 128, _zmark, 0)
    for i in range(128 // 16):
        irow_v[0, pl.ds(i * 16, 16)] = base16 + (i * 16)

    @pl.when(s == 0)
    def _zmacc():
        pltpu.sync_copy(mark_v, macc)

    plsc.subcore_barrier()

    def _chunk(j, _):
        for i in range(128 // 16):
            sl = pl.ds(i * 16, 16)
            cv = col_v[j, sl]
            rv = row_v[j, sl]
            a = plsc.load_gather(dinv_v, [rv])
            b = plsc.load_gather(dinv_v, [cv])
            n = a * ew_v[j, sl] * b
            norm_v[j, sl] = n
            m3 = jnp.logical_or(cv == ACE_IDX, cv == H2_IDX)
            norm3_v[j, sl] = jnp.where(m3, n, 0.0)
            plsc.store_scatter(mark_v, [rv >> 7, rv & 127], ione, mask=m3)
        return 0

    lax.fori_loop(0, NIR, _chunk, 0)
    pltpu.sync_copy(norm_v, out_hbm.at[wid])
    pltpu.sync_copy(norm3_v, out3_hbm.at[wid])
    pltpu.sync_copy(mark_v, macc.at[irow_v.at[0]], add=True)
    plsc.subcore_barrier()

    @pl.when(s == 0)
    def _cpmacc():
        pltpu.sync_copy(macc, outm_hbm.at[c])


@functools.partial(
    pl.kernel,
    out_type=jax.ShapeDtypeStruct((NC, NPAD, D), jnp.float32),
    mesh=_mesh,
    compiler_params=_sc_params,
    scratch_types=[
        pltpu.VMEM((NIR, 128), jnp.int32),
        pltpu.VMEM((NIR, 128), jnp.int32),
        pltpu.VMEM((NIR // NSEG, 128), jnp.float32),
        pltpu.VMEM((1, 128), jnp.int32),
        pltpu.VMEM((CH, D), jnp.float32),
        pltpu.VMEM((CH, D), jnp.float32),
        pltpu.VMEM((CH, D), jnp.float32),
        pltpu.VMEM_SHARED((ANPAD, D), jnp.float32),
        pltpu.SemaphoreType.DMA,
        pltpu.SemaphoreType.DMA,
        pltpu.SemaphoreType.DMA,
        pltpu.SemaphoreType.DMA,
        pltpu.SemaphoreType.DMA,
        pltpu.SemaphoreType.DMA,
    ],
)
def _agg_kernel(hs_hbm, row_hbm, col_hbm, norm_hbm, cnt_hbm, out_hbm,
                row_v, col_v, norm_v, cnt_v, rb0, rb1, rb2, acc,
                g0, g1, g2, s0, s1, s2):
    c = lax.axis_index("c")
    s = lax.axis_index("s")
    wid = s * NC + c
    rbufs = (rb0, rb1, rb2)
    gsems = (g0, g1, g2)
    ssems = (s0, s1, s2)

    pltpu.sync_copy(row_hbm.at[wid], row_v)
    pltpu.sync_copy(col_hbm.at[wid], col_v)
    pltpu.sync_copy(cnt_hbm.at[wid], cnt_v.at[0])

    zero = jnp.zeros((16,), jnp.float32)

    def _zrow(e, _):
        for k in range(D // 16):
            rb0[e, pl.ds(k * 16, 16)] = zero
        return 0

    lax.fori_loop(0, CH, _zrow, 0)
    for t in range(ARPT // CH):
        pltpu.sync_copy(rb0, acc.at[pl.ds(s * ARPT + t * CH, CH)])
    pltpu.sync_copy(rb0.at[pl.ds(0, ARPT % CH)],
                    acc.at[pl.ds(s * ARPT + (ARPT // CH) * CH, ARPT % CH)])
    plsc.subcore_barrier()

    def _ivec(idx_v, j, q):
        flat = j * CH + q * 16
        return idx_v[flat // 128, pl.ds(pl.multiple_of(flat % 128, 16), 16)]

    def _fire_gather(j, l):
        for q in range(CH // 16):
            pltpu.async_copy(hs_hbm.at[_ivec(row_v, j, q)],
                             rbufs[l].at[pl.ds(q * 16, 16)], gsems[l])

    def _wait_gather(j, l):
        for q in range(CH // 16):
            pltpu.make_async_copy(hs_hbm.at[_ivec(row_v, j, q)],
                                  rbufs[l].at[pl.ds(q * 16, 16)],
                                  gsems[l]).wait()

    def _fire_scatter(j, l):
        for q in range(CH // 16):
            pltpu.async_copy(rbufs[l].at[pl.ds(q * 16, 16)],
                             acc.at[_ivec(col_v, j, q)], ssems[l], add=True)

    def _wait_scatter(l):
        zi = jnp.zeros((16,), jnp.int32)
        for q in range(CH // 16):
            pltpu.make_async_copy(rbufs[l].at[pl.ds(q * 16, 16)],
                                  acc.at[zi], ssems[l]).wait()

    def _scale(l, j, seg):
        base = (j - seg * (NCHUNK // NSEG)) * CH

        def _grp(g, _):
            o = base + g * 16
            w16 = norm_v[o // 128, pl.ds(pl.multiple_of(o % 128, 16), 16)]
            rb = rbufs[l]
            for u in range(16):
                e = g * 16 + u
                w = jnp.full((16,), w16[u], jnp.float32)
                for k in range(D // 16):
                    sl = pl.ds(k * 16, 16)
                    rb[e, sl] = rb[e, sl] * w
            return 0

        lax.fori_loop(0, CH // 16, _grp, 0)

    cnt = cnt_v[0, pl.ds(0, 16)][0]
    ng = jnp.maximum((cnt + 3 * CH - 1) // (3 * CH), 1)

    _fire_gather(0, 0)
    _fire_gather(1, 1)

    def _group(g, _):
        seg = g // GPSEG

        @pl.when(g % GPSEG == 0)
        def _stage_norm():
            pltpu.sync_copy(norm_hbm.at[wid, seg], norm_v)

        for l in range(3):
            j = g * 3 + l
            p = (l + 2) % 3

            if l == 0:
                pl.when(g > 0)(lambda: _wait_scatter(p))
            else:
                _wait_scatter(p)

            if l == 0:
                _fire_gather(j + 2, p)
            else:
                pl.when(g < ng - 1)(lambda: _fire_gather(j + 2, p))

            _wait_gather(j, l)
            _scale(l, j, seg)
            _fire_scatter(j, l)
        return 0

    lax.fori_loop(0, ng, _group, 0)
    _wait_scatter(2)

    plsc.subcore_barrier()
    pltpu.sync_copy(acc.at[pl.ds(s * ARPT, ARPT)],
                    out_hbm.at[c, pl.ds(s * ARPT, ARPT)])


A_BASE = (ACE_IDX // 8) * 8
H_BASE = (H2_IDX // 8) * 8
A_TILE = ACE_IDX // ARPT
H_TILE = H2_IDX // ARPT


@functools.partial(
    pl.kernel,
    out_type=jax.ShapeDtypeStruct((NC, 2, 8, D), jnp.float32),
    mesh=_mesh,
    compiler_params=_sc_params,
    scratch_types=[
        pltpu.VMEM((NIR, 128), jnp.int32),
        pltpu.VMEM((NIR, 128), jnp.int32),
        pltpu.VMEM((NIR, 128), jnp.float32),
        pltpu.VMEM((CH, D), jnp.float32),
        pltpu.VMEM_SHARED((ANPAD, D), jnp.float32),
        pltpu.SemaphoreType.DMA,
        pltpu.SemaphoreType.DMA,
    ],
)
def _agg3_kernel(hs_hbm, row_hbm, col_hbm, norm3_hbm, out_hbm,
                 row_v, col_v, norm_v, rb, acc, gsem, ssem):
    c = lax.axis_index("c")
    s = lax.axis_index("s")
    wid = s * NC + c

    pltpu.sync_copy(row_hbm.at[wid], row_v)
    pltpu.sync_copy(col_hbm.at[wid], col_v)
    pltpu.sync_copy(norm3_hbm.at[wid], norm_v)

    zero = jnp.zeros((16,), jnp.float32)

    def _zrow(e, _):
        for k in range(D // 16):
            rb[e, pl.ds(k * 16, 16)] = zero
        return 0

    lax.fori_loop(0, 8, _zrow, 0)

    @pl.when(s == 0)
    def _za():
        pltpu.sync_copy(rb.at[pl.ds(0, 8)], acc.at[pl.ds(A_BASE, 8)])

    @pl.when(s == 1)
    def _zh():
        pltpu.sync_copy(rb.at[pl.ds(0, 8)], acc.at[pl.ds(H_BASE, 8)])

    plsc.subcore_barrier()

    def _ivec(idx_v, j, q):
        flat = j * CH + q * 16
        return idx_v[flat // 128, pl.ds(pl.multiple_of(flat % 128, 16), 16)]

    def _chunk(j, _):
        nz = jnp.zeros((16,), jnp.bool_)
        for q in range(CH // 16):
            nz = jnp.logical_or(nz, _ivec(norm_v, j, q) != 0.0)
        cnt = plsc.all_reduce_population_count(nz)

        @pl.when(cnt[0] > 0)
        def _do():
            for q in range(CH // 16):
                pltpu.async_copy(hs_hbm.at[_ivec(row_v, j, q)],
                                 rb.at[pl.ds(q * 16, 16)], gsem)
            for q in range(CH // 16):
                pltpu.make_async_copy(hs_hbm.at[_ivec(row_v, j, q)],
                                      rb.at[pl.ds(q * 16, 16)], gsem).wait()

            def _grp(g, _):
                o = j * CH + g * 16
                w16 = norm_v[o // 128,
                             pl.ds(pl.multiple_of(o % 128, 16), 16)]
                for u in range(16):
                    e = g * 16 + u
                    w = jnp.full((16,), w16[u], jnp.float32)
                    for k in range(D // 16):
                        sl = pl.ds(k * 16, 16)
                        rb[e, sl] = rb[e, sl] * w
                return 0

            lax.fori_loop(0, CH // 16, _grp, 0)
            for q in range(CH // 16):
                pltpu.async_copy(rb.at[pl.ds(q * 16, 16)],
                                 acc.at[_ivec(col_v, j, q)], ssem, add=True)
            for q in range(CH // 16):
                zi = jnp.zeros((16,), jnp.int32)
                pltpu.make_async_copy(rb.at[pl.ds(q * 16, 16)],
                                      acc.at[zi], ssem).wait()

        return 0

    lax.fori_loop(0, NCHUNK, _chunk, 0)
    plsc.subcore_barrier()

    @pl.when(s == 0)
    def _ca():
        pltpu.sync_copy(acc.at[pl.ds(A_BASE, 8)], out_hbm.at[c, 0])

    @pl.when(s == 1)
    def _ch():
        pltpu.sync_copy(acc.at[pl.ds(H_BASE, 8)], out_hbm.at[c, 1])


@functools.partial(
    pl.kernel,
    out_type=(jax.ShapeDtypeStruct((NW, NIR, 128), jnp.int32),
              jax.ShapeDtypeStruct((NW, NIR, 128), jnp.int32),
              jax.ShapeDtypeStruct((NW, NIR, 128), jnp.float32),
              jax.ShapeDtypeStruct((NW, 128), jnp.int32),
              jax.ShapeDtypeStruct((NC, 128, 128), jnp.float32)),
    mesh=_mesh,
    compiler_params=_sc_params,
    scratch_types=[
        pltpu.VMEM((NIR, 128), jnp.int32),
        pltpu.VMEM((NIR, 128), jnp.int32),
        pltpu.VMEM((NIR, 128), jnp.float32),
        pltpu.VMEM((128, 128), jnp.float32),
        pltpu.VMEM((128, 128), jnp.float32),
        pltpu.VMEM((NIR, 128), jnp.int32),
        pltpu.VMEM((NIR, 128), jnp.int32),
        pltpu.VMEM((NIR, 128), jnp.float32),
        pltpu.VMEM((1, 128), jnp.int32),
        pltpu.VMEM((128, 128), jnp.float32),
        pltpu.VMEM((1, 128), jnp.int32),
        pltpu.VMEM_SHARED((128, 128), jnp.float32),
    ],
)
def _compact_kernel(row_hbm, col_hbm, norm_hbm, markp_hbm,
                    rowc_hbm, colc_hbm, normc_hbm, cnt_hbm, outm_hbm,
                    row_v, col_v, norm_v, m0_v, m1_v,
                    rowc_v, colc_v, normc_v, cnt_v, mark_v, irow_v, macc):
    c = lax.axis_index("c")
    s = lax.axis_index("s")
    wid = s * NC + c

    pltpu.sync_copy(row_hbm.at[wid], row_v)
    pltpu.sync_copy(col_hbm.at[wid], col_v)
    pltpu.sync_copy(norm_hbm.at[wid], norm_v)
    pltpu.sync_copy(markp_hbm.at[0], m0_v)
    pltpu.sync_copy(markp_hbm.at[1], m1_v)

    izero = jnp.zeros((16,), jnp.int32)
    fzero = jnp.zeros((16,), jnp.float32)
    fone = jnp.ones((16,), jnp.float32)
    base16 = jnp.arange(16, dtype=jnp.int32)

    def _zfill(j, _):
        for i in range(128 // 16):
            sl = pl.ds(i * 16, 16)
            rowc_v[j, sl] = izero
            colc_v[j, sl] = izero
            normc_v[j, sl] = fzero
        return 0

    lax.fori_loop(0, NIR, _zfill, 0)

    def _zmark(r, _):
        for i in range(128 // 16):
            mark_v[r, pl.ds(i * 16, 16)] = fzero
        return 0

    lax.fori_loop(0, 128, _zmark, 0)
    for i in range(128 // 16):
        irow_v[0, pl.ds(i * 16, 16)] = base16 + (i * 16)

    @pl.when(s == 0)
    def _zmacc():
        pltpu.sync_copy(mark_v, macc)

    plsc.subcore_barrier()

    def _chunk(j, base):
        for i in range(128 // 16):
            sl = pl.ds(i * 16, 16)
            cv = col_v[j, sl]
            rv = row_v[j, sl]
            nv = norm_v[j, sl]
            hi = cv >> 7
            lo = cv & 127
            f = (plsc.load_gather(m0_v, [hi, lo])
                 + plsc.load_gather(m1_v, [hi, lo]))
            m = f > 0.0
            inc = plsc.cumsum(jnp.where(m, 1, 0))
            pos = base + inc - 1
            phi = pos >> 7
            plo = pos & 127
            plsc.store_scatter(rowc_v, [phi, plo], rv, mask=m)
            plsc.store_scatter(colc_v, [phi, plo], cv, mask=m)
            plsc.store_scatter(normc_v, [phi, plo], nv, mask=m)
            plsc.store_scatter(mark_v, [rv >> 7, rv & 127], fone, mask=m)
            base = base + inc[15]
        return base

    cnt = lax.fori_loop(0, NIR, _chunk, 0)
    for i in range(128 // 16):
        cnt_v[0, pl.ds(i * 16, 16)] = jnp.full((16,), cnt, jnp.int32)
    pltpu.sync_copy(rowc_v, rowc_hbm.at[wid])
    pltpu.sync_copy(colc_v, colc_hbm.at[wid])
    pltpu.sync_copy(normc_v, normc_hbm.at[wid])
    pltpu.sync_copy(cnt_v.at[0], cnt_hbm.at[wid])
    pltpu.sync_copy(mark_v, macc.at[irow_v.at[0]], add=True)
    plsc.subcore_barrier()

    @pl.when(s == 0)
    def _cpmacc():
        pltpu.sync_copy(macc, outm_hbm.at[c])


@functools.partial(
    pl.kernel,
    out_type=jax.ShapeDtypeStruct((NC, NPAD, D), jnp.float32),
    mesh=_mesh,
    compiler_params=_sc_params,
    scratch_types=[
        pltpu.VMEM((NIR, 128), jnp.int32),
        pltpu.VMEM((NIR, 128), jnp.int32),
        pltpu.VMEM((NIR, 128), jnp.float32),
        pltpu.VMEM((1, 128), jnp.int32),
        pltpu.VMEM((CH, D), jnp.float32),
        pltpu.VMEM_SHARED((ANPAD, D), jnp.float32),
        pltpu.SemaphoreType.DMA,
        pltpu.SemaphoreType.DMA,
    ],
)
def _agg2_kernel(hs_hbm, row_hbm, col_hbm, norm_hbm, cnt_hbm, out_hbm,
                 row_v, col_v, norm_v, cnt_v, rb, acc, gsem, ssem):
    c = lax.axis_index("c")
    s = lax.axis_index("s")
    wid = s * NC + c

    pltpu.sync_copy(row_hbm.at[wid], row_v)
    pltpu.sync_copy(col_hbm.at[wid], col_v)
    pltpu.sync_copy(norm_hbm.at[wid], norm_v)
    pltpu.sync_copy(cnt_hbm.at[wid], cnt_v.at[0])

    zero = jnp.zeros((16,), jnp.float32)

    def _zrow(e, _):
        for k in range(D // 16):
            rb[e, pl.ds(k * 16, 16)] = zero
        return 0

    lax.fori_loop(0, CH, _zrow, 0)
    for t in range(ARPT // CH):
        pltpu.sync_copy(rb, acc.at[pl.ds(s * ARPT + t * CH, CH)])
    pltpu.sync_copy(rb.at[pl.ds(0, ARPT % CH)],
                    acc.at[pl.ds(s * ARPT + (ARPT // CH) * CH, ARPT % CH)])
    plsc.subcore_barrier()

    def _ivec(idx_v, j, q):
        flat = j * CH + q * 16
        return idx_v[flat // 128, pl.ds(pl.multiple_of(flat % 128, 16), 16)]

    cnt = cnt_v[0, pl.ds(0, 16)][0]
    nch = (cnt + CH - 1) // CH

    def _chunk(j, _):
        for q in range(CH // 16):
            pltpu.async_copy(hs_hbm.at[_ivec(row_v, j, q)],
                             rb.at[pl.ds(q * 16, 16)], gsem)
        for q in range(CH // 16):
            pltpu.make_async_copy(hs_hbm.at[_ivec(row_v, j, q)],
                                  rb.at[pl.ds(q * 16, 16)], gsem).wait()

        def _grp(g, _):
            o = j * CH + g * 16
            w16 = norm_v[o // 128, pl.ds(pl.multiple_of(o % 128, 16), 16)]
            for u in range(16):
                e = g * 16 + u
                w = jnp.full((16,), w16[u], jnp.float32)
                for k in range(D // 16):
                    sl = pl.ds(k * 16, 16)
                    rb[e, sl] = rb[e, sl] * w
            return 0

        lax.fori_loop(0, CH // 16, _grp, 0)
        for q in range(CH // 16):
            pltpu.async_copy(rb.at[pl.ds(q * 16, 16)],
                             acc.at[_ivec(col_v, j, q)], ssem, add=True)
        zi = jnp.zeros((16,), jnp.int32)
        for q in range(CH // 16):
            pltpu.make_async_copy(rb.at[pl.ds(q * 16, 16)],
                                  acc.at[zi], ssem).wait()
        return 0

    lax.fori_loop(0, nch, _chunk, 0)
    plsc.subcore_barrier()
    pltpu.sync_copy(acc.at[pl.ds(s * ARPT, ARPT)],
                    out_hbm.at[c, pl.ds(s * ARPT, ARPT)])


def _dinv_body(dp_ref, o_ref):
    deg = jnp.sum(dp_ref[...], axis=0)
    o_ref[...] = lax.rsqrt(deg)


_dinv_call = pl.pallas_call(
    _dinv_body,
    out_shape=jax.ShapeDtypeStruct((NPAD // D, D), jnp.float32),
)


def _mm_body(x_ref, w_ref, o_ref):
    o_ref[...] = jnp.dot(x_ref[...], w_ref[...],
                         preferred_element_type=jnp.float32)


_mm_call = pl.pallas_call(
    _mm_body,
    grid=(NBLK,),
    in_specs=[
        pl.BlockSpec((MBLK, D), lambda i: (i, 0)),
        pl.BlockSpec((D, D), lambda i: (0, 0)),
    ],
    out_specs=pl.BlockSpec((MBLK, D), lambda i: (i, 0)),
    out_shape=jax.ShapeDtypeStruct((NPAD, D), jnp.float32),
)


def _layer_body(p_ref, b_ref, w_ref, o_ref):
    t = jax.nn.relu(p_ref[0] + p_ref[1] + b_ref[...])
    o_ref[...] = jnp.dot(t, w_ref[...], preferred_element_type=jnp.float32)


_layer_call = pl.pallas_call(
    _layer_body,
    grid=(NBLK,),
    in_specs=[
        pl.BlockSpec((NC, MBLK, D), lambda i: (0, i, 0)),
        pl.BlockSpec((1, D), lambda i: (0, 0)),
        pl.BlockSpec((D, D), lambda i: (0, 0)),
    ],
    out_specs=pl.BlockSpec((MBLK, D), lambda i: (i, 0)),
    out_shape=jax.ShapeDtypeStruct((NPAD, D), jnp.float32),
)


def _head_body(pr_ref, b3_ref, wh_ref, bh_ref, wag_ref, bag_ref, o_ref):
    t = jax.nn.relu(pr_ref[0] + pr_ref[1] + b3_ref[...])
    h = jax.nn.relu(jnp.dot(t, wh_ref[...],
                            preferred_element_type=jnp.float32) + bh_ref[...])
    o_ref[...] = jnp.dot(h, wag_ref[...],
                         preferred_element_type=jnp.float32) + bag_ref[...]


_head_call = pl.pallas_call(
    _head_body,
    out_shape=jax.ShapeDtypeStruct((2, 2), jnp.float32),
)


def kernel(x, edge_index, edge_weight, W1, b1, W2, b2, W3, b3,
           Wh, bh, Wa, ba, Wg, bg):
    f32 = jnp.float32
    row = edge_index[0].astype(jnp.int32)
    col = edge_index[1].astype(jnp.int32)
    ew = edge_weight.astype(f32)

    loop = jnp.arange(N_NODES, dtype=jnp.int32)
    npad_e = E_PAD - N_EDGES - N_NODES
    pad_idx = jnp.arange(npad_e, dtype=jnp.int32) % N_NODES
    eshape = (NW, NIR, 128)
    row_all = jnp.concatenate([row, loop, pad_idx]).reshape(eshape)
    col_all = jnp.concatenate([col, loop, pad_idx]).reshape(eshape)
    ew_all = jnp.concatenate(
        [ew, jnp.ones((N_NODES,), f32), jnp.zeros((npad_e,), f32)]
    ).reshape(eshape)

    dp = _deg_kernel(col_all, ew_all)
    dinv2d = _dinv_call(dp.reshape(NW, NPAD // D, D))
    norm, norm3, markp2 = _norm_kernel(row_all, col_all, ew_all,
                                       dinv2d.reshape(NPAD))
    rowc2, colc2, normc2, cnt2, markp1 = _compact_kernel(
        row_all, col_all, norm, markp2)
    rowc1, colc1, normc1, cnt1, _mp0 = _compact_kernel(
        row_all, col_all, norm, markp1)
    normc1_seg = normc1.reshape(NW, NSEG, NIR // NSEG, 128)

    x_pad = jnp.zeros((NPAD, D), f32).at[:N_NODES].set(x.astype(f32))

    hs = _mm_call(x_pad, W1)
    p = _agg_kernel(hs, rowc1, colc1, normc1_seg, cnt1)
    hs = _layer_call(p, b1.reshape(1, D), W2)
    p = _agg2_kernel(hs, rowc2, colc2, normc2, cnt2)
    hs = _layer_call(p, b2.reshape(1, D), W3)
    p3 = _agg3_kernel(hs, row_all, col_all, norm3)

    pr = jnp.stack([p3[:, 0, ACE_IDX % 8, :], p3[:, 1, H2_IDX % 8, :]],
                   axis=1)
    wag = jnp.concatenate([Wa, Wg], axis=1)
    bag = jnp.stack([ba[0], bg[0]]).reshape(1, 2)
    out22 = _head_call(pr, b3.reshape(1, D), Wh, bh.reshape(1, D), wag, bag)
    return jnp.stack([out22[0, 0], out22[1, 1]])


# speedup vs baseline: 1.1416x; 1.1416x over previous
import functools

import jax
import jax.numpy as jnp
from jax import lax
from jax.experimental import pallas as pl
from jax.experimental.pallas import tpu as pltpu
from jax.experimental.pallas import tpu_sc as plsc

N_NODES = 10000
D = 128
N_EDGES = 320000
ACE_IDX = 1234
H2_IDX = 5678

NC = 2
NS = 16
NW = NC * NS
CH = 48
NCHUNK = 216
NG = NCHUNK // 3
NSEG = 3
GPSEG = NG // NSEG
NIR = 81
EPW = NCHUNK * CH
E_PAD = EPW * NW
NPAD = 10240
ANPAD = 10112
ARPT = ANPAD // NS
MBLK = 1024
NBLK = NPAD // MBLK

_mesh = plsc.VectorSubcoreMesh(core_axis_name="c", subcore_axis_name="s")
_sc_params = pltpu.CompilerParams(needs_layout_passes=False)


@functools.partial(
    pl.kernel,
    out_type=jax.ShapeDtypeStruct((NW, NPAD), jnp.float32),
    mesh=_mesh,
    compiler_params=_sc_params,
    scratch_types=[
        pltpu.VMEM((NIR, 128), jnp.int32),
        pltpu.VMEM((NIR, 128), jnp.float32),
        pltpu.VMEM((NPAD,), jnp.float32),
    ],
)
def _deg_kernel(col_hbm, ew_hbm, out_hbm, col_v, ew_v, deg_v):
    c = lax.axis_index("c")
    s = lax.axis_index("s")
    wid = s * NC + c

    pltpu.sync_copy(col_hbm.at[wid], col_v)
    pltpu.sync_copy(ew_hbm.at[wid], ew_v)

    zero = jnp.zeros((16,), jnp.float32)

    def _zero(i, _):
        deg_v[pl.ds(pl.multiple_of(i * 16, 16), 16)] = zero
        return 0

    lax.fori_loop(0, NPAD // 16, _zero, 0)

    def _chunk(j, _):
        for i in range(128 // 16):
            sl = pl.ds(i * 16, 16)
            plsc.addupdate_scatter(deg_v, [col_v[j, sl]], ew_v[j, sl])
        return 0

    lax.fori_loop(0, NIR, _chunk, 0)
    pltpu.sync_copy(deg_v, out_hbm.at[wid])


@functools.partial(
    pl.kernel,
    out_type=(jax.ShapeDtypeStruct((NW, NIR, 128), jnp.float32),
              jax.ShapeDtypeStruct((NW, NIR, 128), jnp.float32),
              jax.ShapeDtypeStruct((NC, 128, 128), jnp.float32)),
    mesh=_mesh,
    compiler_params=_sc_params,
    scratch_types=[
        pltpu.VMEM((NIR, 128), jnp.int32),
        pltpu.VMEM((NIR, 128), jnp.int32),
        pltpu.VMEM((NIR, 128), jnp.float32),
        pltpu.VMEM((NIR, 128), jnp.float32),
        pltpu.VMEM((NIR, 128), jnp.float32),
        pltpu.VMEM((NPAD,), jnp.float32),
        pltpu.VMEM((128, 128), jnp.float32),
        pltpu.VMEM((1, 128), jnp.int32),
        pltpu.VMEM_SHARED((128, 128), jnp.float32),
    ],
)
def _norm_kernel(row_hbm, col_hbm, ew_hbm, dinv_hbm, out_hbm, out3_hbm,
                 outm_hbm, row_v, col_v, ew_v, norm_v, norm3_v, dinv_v,
                 mark_v, irow_v, macc):
    c = lax.axis_index("c")
    s = lax.axis_index("s")
    wid = s * NC + c

    pltpu.sync_copy(row_hbm.at[wid], row_v)
    pltpu.sync_copy(col_hbm.at[wid], col_v)
    pltpu.sync_copy(ew_hbm.at[wid], ew_v)
    pltpu.sync_copy(dinv_hbm, dinv_v)

    zero = jnp.zeros((16,), jnp.float32)
    ione = jnp.ones((16,), jnp.float32)
    base16 = jnp.arange(16, dtype=jnp.int32)

    def _zmark(r, _):
        for i in range(128 // 16):
            mark_v[r, pl.ds(i * 16, 16)] = zero
        return 0

    lax.fori_loop(0, 128, _zmark, 0)
    for i in range(128 // 16):
        irow_v[0, pl.ds(i * 16, 16)] = base16 + (i * 16)

    @pl.when(s == 0)
    def _zmacc():
        pltpu.sync_copy(mark_v, macc)

    plsc.subcore_barrier()

    def _chunk(j, _):
        for i in range(128 // 16):
            sl = pl.ds(i * 16, 16)
            cv = col_v[j, sl]
            rv = row_v[j, sl]
            a = plsc.load_gather(dinv_v, [rv])
            b = plsc.load_gather(dinv_v, [cv])
            n = a * ew_v[j, sl] * b
            norm_v[j, sl] = n
            m3 = jnp.logical_or(cv == ACE_IDX, cv == H2_IDX)
            norm3_v[j, sl] = jnp.where(m3, n, 0.0)
            plsc.store_scatter(mark_v, [rv >> 7, rv & 127], ione, mask=m3)
        return 0

    lax.fori_loop(0, NIR, _chunk, 0)
    pltpu.sync_copy(norm_v, out_hbm.at[wid])
    pltpu.sync_copy(norm3_v, out3_hbm.at[wid])
    pltpu.sync_copy(mark_v, macc.at[irow_v.at[0]], add=True)
    plsc.subcore_barrier()

    @pl.when(s == 0)
    def _cpmacc():
        pltpu.sync_copy(macc, outm_hbm.at[c])


@functools.partial(
    pl.kernel,
    out_type=jax.ShapeDtypeStruct((NC, NPAD, D), jnp.float32),
    mesh=_mesh,
    compiler_params=_sc_params,
    scratch_types=[
        pltpu.VMEM((NIR, 128), jnp.int32),
        pltpu.VMEM((NIR, 128), jnp.int32),
        pltpu.VMEM((NIR // NSEG, 128), jnp.float32),
        pltpu.VMEM((CH, D), jnp.float32),
        pltpu.VMEM((CH, D), jnp.float32),
        pltpu.VMEM((CH, D), jnp.float32),
        pltpu.VMEM_SHARED((ANPAD, D), jnp.float32),
        pltpu.SemaphoreType.DMA,
        pltpu.SemaphoreType.DMA,
        pltpu.SemaphoreType.DMA,
        pltpu.SemaphoreType.DMA,
        pltpu.SemaphoreType.DMA,
        pltpu.SemaphoreType.DMA,
    ],
)
def _agg_kernel(hs_hbm, row_hbm, col_hbm, norm_hbm, out_hbm,
                row_v, col_v, norm_v, rb0, rb1, rb2, acc,
                g0, g1, g2, s0, s1, s2):
    c = lax.axis_index("c")
    s = lax.axis_index("s")
    wid = s * NC + c
    rbufs = (rb0, rb1, rb2)
    gsems = (g0, g1, g2)
    ssems = (s0, s1, s2)

    pltpu.async_copy(row_hbm.at[wid], row_v, g0)
    pltpu.async_copy(col_hbm.at[wid], col_v, g1)

    zero = jnp.zeros((16,), jnp.float32)

    def _zrow(e, _):
        for k in range(D // 16):
            rb1[e, pl.ds(k * 16, 16)] = zero
        return 0

    lax.fori_loop(0, CH, _zrow, 0)
    for t in range(ARPT // CH):
        pltpu.async_copy(rb1, acc.at[pl.ds(s * ARPT + t * CH, CH)], s0)
    pltpu.async_copy(rb1.at[pl.ds(0, ARPT % CH)],
                     acc.at[pl.ds(s * ARPT + (ARPT // CH) * CH, ARPT % CH)],
                     s0)
    for t in range(ARPT // CH):
        pltpu.make_async_copy(rb1, acc.at[pl.ds(s * ARPT + t * CH, CH)],
                              s0).wait()
    pltpu.make_async_copy(rb1.at[pl.ds(0, ARPT % CH)],
                          acc.at[pl.ds(s * ARPT, ARPT % CH)], s0).wait()
    pltpu.make_async_copy(row_hbm.at[wid], row_v, g0).wait()
    pltpu.make_async_copy(col_hbm.at[wid], col_v, g1).wait()
    plsc.subcore_barrier()

    def _ivec(idx_v, j, q):
        flat = j * CH + q * 16
        return idx_v[flat // 128, pl.ds(pl.multiple_of(flat % 128, 16), 16)]

    def _fire_gather(j, l):
        for q in range(CH // 16):
            pltpu.async_copy(hs_hbm.at[_ivec(row_v, j, q)],
                             rbufs[l].at[pl.ds(q * 16, 16)], gsems[l])

    def _wait_gather(j, l):
        for q in range(CH // 16):
            pltpu.make_async_copy(hs_hbm.at[_ivec(row_v, j, q)],
                                  rbufs[l].at[pl.ds(q * 16, 16)],
                                  gsems[l]).wait()

    def _fire_scatter(j, l):
        for q in range(CH // 16):
            pltpu.async_copy(rbufs[l].at[pl.ds(q * 16, 16)],
                             acc.at[_ivec(col_v, j, q)], ssems[l], add=True)

    def _wait_scatter(l):
        zi = jnp.zeros((16,), jnp.int32)
        for q in range(CH // 16):
            pltpu.make_async_copy(rbufs[l].at[pl.ds(q * 16, 16)],
                                  acc.at[zi], ssems[l]).wait()

    def _scale(l, j, seg):
        base = (j - seg * (NCHUNK // NSEG)) * CH

        def _grp(g, _):
            o = base + g * 16
            w16 = norm_v[o // 128, pl.ds(pl.multiple_of(o % 128, 16), 16)]
            rb = rbufs[l]
            for u in range(16):
                e = g * 16 + u
                w = jnp.full((16,), w16[u], jnp.float32)
                for k in range(D // 16):
                    sl = pl.ds(k * 16, 16)
                    rb[e, sl] = rb[e, sl] * w
            return 0

        lax.fori_loop(0, CH // 16, _grp, 0)

    _fire_gather(0, 0)
    _fire_gather(1, 1)

    def _group(g, _):
        seg = g // GPSEG

        @pl.when(g % GPSEG == 0)
        def _stage_norm():
            pltpu.sync_copy(norm_hbm.at[wid, seg], norm_v)

        for l in range(3):
            j = g * 3 + l
            p = (l + 2) % 3

            if l == 0:
                pl.when(g > 0)(lambda: _wait_scatter(p))
            else:
                _wait_scatter(p)

            if l == 0:
                _fire_gather(j + 2, p)
            else:
                pl.when(g < NG - 1)(lambda: _fire_gather(j + 2, p))

            _wait_gather(j, l)
            _scale(l, j, seg)
            _fire_scatter(j, l)
        return 0

    lax.fori_loop(0, NG, _group, 0)
    _wait_scatter(2)

    plsc.subcore_barrier()
    pltpu.sync_copy(acc.at[pl.ds(s * ARPT, ARPT)],
                    out_hbm.at[c, pl.ds(s * ARPT, ARPT)])


A_BASE = (ACE_IDX // 8) * 8
H_BASE = (H2_IDX // 8) * 8
A_TILE = ACE_IDX // ARPT
H_TILE = H2_IDX // ARPT


@functools.partial(
    pl.kernel,
    out_type=jax.ShapeDtypeStruct((NC, 2, 8, D), jnp.float32),
    mesh=_mesh,
    compiler_params=_sc_params,
    scratch_types=[
        pltpu.VMEM((NIR, 128), jnp.int32),
        pltpu.VMEM((NIR, 128), jnp.int32),
        pltpu.VMEM((NIR, 128), jnp.float32),
        pltpu.VMEM((CH, D), jnp.float32),
        pltpu.VMEM_SHARED((ANPAD, D), jnp.float32),
        pltpu.SemaphoreType.DMA,
        pltpu.SemaphoreType.DMA,
    ],
)
def _agg3_kernel(hs_hbm, row_hbm, col_hbm, norm3_hbm, out_hbm,
                 row_v, col_v, norm_v, rb, acc, gsem, ssem):
    c = lax.axis_index("c")
    s = lax.axis_index("s")
    wid = s * NC + c

    pltpu.async_copy(row_hbm.at[wid], row_v, gsem)
    pltpu.async_copy(col_hbm.at[wid], col_v, gsem)
    pltpu.async_copy(norm3_hbm.at[wid], norm_v, gsem)
    pltpu.make_async_copy(row_hbm.at[wid], row_v, gsem).wait()
    pltpu.make_async_copy(col_hbm.at[wid], col_v, gsem).wait()
    pltpu.make_async_copy(norm3_hbm.at[wid], norm_v, gsem).wait()

    zero = jnp.zeros((16,), jnp.float32)

    def _zrow(e, _):
        for k in range(D // 16):
            rb[e, pl.ds(k * 16, 16)] = zero
        return 0

    lax.fori_loop(0, 8, _zrow, 0)

    @pl.when(s == 0)
    def _za():
        pltpu.sync_copy(rb.at[pl.ds(0, 8)], acc.at[pl.ds(A_BASE, 8)])

    @pl.when(s == 1)
    def _zh():
        pltpu.sync_copy(rb.at[pl.ds(0, 8)], acc.at[pl.ds(H_BASE, 8)])

    plsc.subcore_barrier()

    def _ivec(idx_v, j, q):
        flat = j * CH + q * 16
        return idx_v[flat // 128, pl.ds(pl.multiple_of(flat % 128, 16), 16)]

    def _chunk(j, _):
        nz = jnp.zeros((16,), jnp.bool_)
        for q in range(CH // 16):
            nz = jnp.logical_or(nz, _ivec(norm_v, j, q) != 0.0)
        cnt = plsc.all_reduce_population_count(nz)

        @pl.when(cnt[0] > 0)
        def _do():
            for q in range(CH // 16):
                pltpu.async_copy(hs_hbm.at[_ivec(row_v, j, q)],
                                 rb.at[pl.ds(q * 16, 16)], gsem)
            for q in range(CH // 16):
                pltpu.make_async_copy(hs_hbm.at[_ivec(row_v, j, q)],
                                      rb.at[pl.ds(q * 16, 16)], gsem).wait()

            def _grp(g, _):
                o = j * CH + g * 16
                w16 = norm_v[o // 128,
                             pl.ds(pl.multiple_of(o % 128, 16), 16)]
                for u in range(16):
                    e = g * 16 + u
                    w = jnp.full((16,), w16[u], jnp.float32)
                    for k in range(D // 16):
                        sl = pl.ds(k * 16, 16)
                        rb[e, sl] = rb[e, sl] * w
                return 0

            lax.fori_loop(0, CH // 16, _grp, 0)
            for q in range(CH // 16):
                pltpu.async_copy(rb.at[pl.ds(q * 16, 16)],
                                 acc.at[_ivec(col_v, j, q)], ssem, add=True)
            for q in range(CH // 16):
                zi = jnp.zeros((16,), jnp.int32)
                pltpu.make_async_copy(rb.at[pl.ds(q * 16, 16)],
                                      acc.at[zi], ssem).wait()

        return 0

    lax.fori_loop(0, NCHUNK, _chunk, 0)
    plsc.subcore_barrier()

    @pl.when(s == 0)
    def _ca():
        pltpu.sync_copy(acc.at[pl.ds(A_BASE, 8)], out_hbm.at[c, 0])

    @pl.when(s == 1)
    def _ch():
        pltpu.sync_copy(acc.at[pl.ds(H_BASE, 8)], out_hbm.at[c, 1])


@functools.partial(
    pl.kernel,
    out_type=(jax.ShapeDtypeStruct((NW, NIR, 128), jnp.int32),
              jax.ShapeDtypeStruct((NW, NIR, 128), jnp.int32),
              jax.ShapeDtypeStruct((NW, NIR, 128), jnp.float32),
              jax.ShapeDtypeStruct((NW, 128), jnp.int32)),
    mesh=_mesh,
    compiler_params=_sc_params,
    scratch_types=[
        pltpu.VMEM((NIR, 128), jnp.int32),
        pltpu.VMEM((NIR, 128), jnp.int32),
        pltpu.VMEM((NIR, 128), jnp.float32),
        pltpu.VMEM((128, 128), jnp.float32),
        pltpu.VMEM((128, 128), jnp.float32),
        pltpu.VMEM((NIR, 128), jnp.int32),
        pltpu.VMEM((NIR, 128), jnp.int32),
        pltpu.VMEM((NIR, 128), jnp.float32),
        pltpu.VMEM((1, 128), jnp.int32),
    ],
)
def _mask2_kernel(row_hbm, col_hbm, norm_hbm, markp_hbm,
                  rowc_hbm, colc_hbm, normc_hbm, cnt_hbm,
                  row_v, col_v, norm_v, m0_v, m1_v,
                  rowc_v, colc_v, normc_v, cnt_v):
    c = lax.axis_index("c")
    s = lax.axis_index("s")
    wid = s * NC + c

    pltpu.sync_copy(row_hbm.at[wid], row_v)
    pltpu.sync_copy(col_hbm.at[wid], col_v)
    pltpu.sync_copy(norm_hbm.at[wid], norm_v)
    pltpu.sync_copy(markp_hbm.at[0], m0_v)
    pltpu.sync_copy(markp_hbm.at[1], m1_v)

    izero = jnp.zeros((16,), jnp.int32)
    fzero = jnp.zeros((16,), jnp.float32)

    def _zfill(j, _):
        for i in range(128 // 16):
            sl = pl.ds(i * 16, 16)
            rowc_v[j, sl] = izero
            colc_v[j, sl] = izero
            normc_v[j, sl] = fzero
        return 0

    lax.fori_loop(0, NIR, _zfill, 0)

    def _chunk(j, base):
        for i in range(128 // 16):
            sl = pl.ds(i * 16, 16)
            cv = col_v[j, sl]
            rv = row_v[j, sl]
            nv = norm_v[j, sl]
            hi = cv >> 7
            lo = cv & 127
            f = (plsc.load_gather(m0_v, [hi, lo])
                 + plsc.load_gather(m1_v, [hi, lo]))
            m = f > 0.0
            inc = plsc.cumsum(jnp.where(m, 1, 0))
            pos = base + inc - 1
            phi = pos >> 7
            plo = pos & 127
            plsc.store_scatter(rowc_v, [phi, plo], rv, mask=m)
            plsc.store_scatter(colc_v, [phi, plo], cv, mask=m)
            plsc.store_scatter(normc_v, [phi, plo], nv, mask=m)
            base = base + inc[15]
        return base

    cnt = lax.fori_loop(0, NIR, _chunk, 0)
    for i in range(128 // 16):
        cnt_v[0, pl.ds(i * 16, 16)] = jnp.full((16,), cnt, jnp.int32)
    pltpu.sync_copy(rowc_v, rowc_hbm.at[wid])
    pltpu.sync_copy(colc_v, colc_hbm.at[wid])
    pltpu.sync_copy(normc_v, normc_hbm.at[wid])
    pltpu.sync_copy(cnt_v.at[0], cnt_hbm.at[wid])


@functools.partial(
    pl.kernel,
    out_type=jax.ShapeDtypeStruct((NC, NPAD, D), jnp.float32),
    mesh=_mesh,
    compiler_params=_sc_params,
    scratch_types=[
        pltpu.VMEM((NIR, 128), jnp.int32),
        pltpu.VMEM((NIR, 128), jnp.int32),
        pltpu.VMEM((NIR, 128), jnp.float32),
        pltpu.VMEM((1, 128), jnp.int32),
        pltpu.VMEM((CH, D), jnp.float32),
        pltpu.VMEM_SHARED((ANPAD, D), jnp.float32),
        pltpu.SemaphoreType.DMA,
        pltpu.SemaphoreType.DMA,
    ],
)
def _agg2_kernel(hs_hbm, row_hbm, col_hbm, norm_hbm, cnt_hbm, out_hbm,
                 row_v, col_v, norm_v, cnt_v, rb, acc, gsem, ssem):
    c = lax.axis_index("c")
    s = lax.axis_index("s")
    wid = s * NC + c

    pltpu.async_copy(row_hbm.at[wid], row_v, gsem)
    pltpu.async_copy(col_hbm.at[wid], col_v, gsem)
    pltpu.async_copy(norm_hbm.at[wid], norm_v, gsem)
    pltpu.async_copy(cnt_hbm.at[wid], cnt_v.at[0], gsem)

    zero = jnp.zeros((16,), jnp.float32)

    def _zrow(e, _):
        for k in range(D // 16):
            rb[e, pl.ds(k * 16, 16)] = zero
        return 0

    lax.fori_loop(0, CH, _zrow, 0)
    pltpu.make_async_copy(row_hbm.at[wid], row_v, gsem).wait()
    pltpu.make_async_copy(col_hbm.at[wid], col_v, gsem).wait()
    pltpu.make_async_copy(norm_hbm.at[wid], norm_v, gsem).wait()
    pltpu.make_async_copy(cnt_hbm.at[wid], cnt_v.at[0], gsem).wait()
    for t in range(ARPT // CH):
        pltpu.async_copy(rb, acc.at[pl.ds(s * ARPT + t * CH, CH)], ssem)
    pltpu.async_copy(rb.at[pl.ds(0, ARPT % CH)],
                     acc.at[pl.ds(s * ARPT + (ARPT // CH) * CH, ARPT % CH)],
                     ssem)
    for t in range(ARPT // CH):
        pltpu.make_async_copy(rb, acc.at[pl.ds(s * ARPT + t * CH, CH)],
                              ssem).wait()
    pltpu.make_async_copy(rb.at[pl.ds(0, ARPT % CH)],
                          acc.at[pl.ds(s * ARPT, ARPT % CH)], ssem).wait()
    plsc.subcore_barrier()

    def _ivec(idx_v, j, q):
        flat = j * CH + q * 16
        return idx_v[flat // 128, pl.ds(pl.multiple_of(flat % 128, 16), 16)]

    cnt = cnt_v[0, pl.ds(0, 16)][0]
    nch = (cnt + CH - 1) // CH

    def _chunk(j, _):
        for q in range(CH // 16):
            pltpu.async_copy(hs_hbm.at[_ivec(row_v, j, q)],
                             rb.at[pl.ds(q * 16, 16)], gsem)
        for q in range(CH // 16):
            pltpu.make_async_copy(hs_hbm.at[_ivec(row_v, j, q)],
                                  rb.at[pl.ds(q * 16, 16)], gsem).wait()

        def _grp(g, _):
            o = j * CH + g * 16
            w16 = norm_v[o // 128, pl.ds(pl.multiple_of(o % 128, 16), 16)]
            for u in range(16):
                e = g * 16 + u
                w = jnp.full((16,), w16[u], jnp.float32)
                for k in range(D // 16):
                    sl = pl.ds(k * 16, 16)
                    rb[e, sl] = rb[e, sl] * w
            return 0

        lax.fori_loop(0, CH // 16, _grp, 0)
        for q in range(CH // 16):
            pltpu.async_copy(rb.at[pl.ds(q * 16, 16)],
                             acc.at[_ivec(col_v, j, q)], ssem, add=True)
        zi = jnp.zeros((16,), jnp.int32)
        for q in range(CH // 16):
            pltpu.make_async_copy(rb.at[pl.ds(q * 16, 16)],
                                  acc.at[zi], ssem).wait()
        return 0

    lax.fori_loop(0, nch, _chunk, 0)
    plsc.subcore_barrier()
    pltpu.sync_copy(acc.at[pl.ds(s * ARPT, ARPT)],
                    out_hbm.at[c, pl.ds(s * ARPT, ARPT)])


def _dinv_body(dp_ref, o_ref):
    deg = jnp.sum(dp_ref[...], axis=0)
    o_ref[...] = lax.rsqrt(deg)


_dinv_call = pl.pallas_call(
    _dinv_body,
    out_shape=jax.ShapeDtypeStruct((NPAD // D, D), jnp.float32),
)


def _mm_body(x_ref, w_ref, o_ref):
    o_ref[...] = jnp.dot(x_ref[...], w_ref[...],
                         preferred_element_type=jnp.float32)


_mm_call = pl.pallas_call(
    _mm_body,
    grid=(NBLK,),
    in_specs=[
        pl.BlockSpec((MBLK, D), lambda i: (i, 0)),
        pl.BlockSpec((D, D), lambda i: (0, 0)),
    ],
    out_specs=pl.BlockSpec((MBLK, D), lambda i: (i, 0)),
    out_shape=jax.ShapeDtypeStruct((NPAD, D), jnp.float32),
)


def _layer_body(p_ref, b_ref, w_ref, o_ref):
    t = jax.nn.relu(p_ref[0] + p_ref[1] + b_ref[...])
    o_ref[...] = jnp.dot(t, w_ref[...], preferred_element_type=jnp.float32)


_layer_call = pl.pallas_call(
    _layer_body,
    grid=(NBLK,),
    in_specs=[
        pl.BlockSpec((NC, MBLK, D), lambda i: (0, i, 0)),
        pl.BlockSpec((1, D), lambda i: (0, 0)),
        pl.BlockSpec((D, D), lambda i: (0, 0)),
    ],
    out_specs=pl.BlockSpec((MBLK, D), lambda i: (i, 0)),
    out_shape=jax.ShapeDtypeStruct((NPAD, D), jnp.float32),
)


def _head_body(pr_ref, b3_ref, wh_ref, bh_ref, wag_ref, bag_ref, o_ref):
    t = jax.nn.relu(pr_ref[0] + pr_ref[1] + b3_ref[...])
    h = jax.nn.relu(jnp.dot(t, wh_ref[...],
                            preferred_element_type=jnp.float32) + bh_ref[...])
    o_ref[...] = jnp.dot(h, wag_ref[...],
                         preferred_element_type=jnp.float32) + bag_ref[...]


_head_call = pl.pallas_call(
    _head_body,
    out_shape=jax.ShapeDtypeStruct((2, 2), jnp.float32),
)


def kernel(x, edge_index, edge_weight, W1, b1, W2, b2, W3, b3,
           Wh, bh, Wa, ba, Wg, bg):
    f32 = jnp.float32
    row = edge_index[0].astype(jnp.int32)
    col = edge_index[1].astype(jnp.int32)
    ew = edge_weight.astype(f32)

    loop = jnp.arange(N_NODES, dtype=jnp.int32)
    npad_e = E_PAD - N_EDGES - N_NODES
    pad_idx = jnp.arange(npad_e, dtype=jnp.int32) % N_NODES
    eshape = (NW, NIR, 128)
    row_all = jnp.concatenate([row, loop, pad_idx]).reshape(eshape)
    col_all = jnp.concatenate([col, loop, pad_idx]).reshape(eshape)
    ew_all = jnp.concatenate(
        [ew, jnp.ones((N_NODES,), f32), jnp.zeros((npad_e,), f32)]
    ).reshape(eshape)

    dp = _deg_kernel(col_all, ew_all)
    dinv2d = _dinv_call(dp.reshape(NW, NPAD // D, D))
    norm, norm3, markp = _norm_kernel(row_all, col_all, ew_all,
                                      dinv2d.reshape(NPAD))
    rowc, colc, normc, cnt2 = _mask2_kernel(row_all, col_all, norm, markp)
    norm_seg = norm.reshape(NW, NSEG, NIR // NSEG, 128)

    x_pad = jnp.zeros((NPAD, D), f32).at[:N_NODES].set(x.astype(f32))

    hs = _mm_call(x_pad, W1)
    p = _agg_kernel(hs, row_all, col_all, norm_seg)
    hs = _layer_call(p, b1.reshape(1, D), W2)
    p = _agg2_kernel(hs, rowc, colc, normc, cnt2)
    hs = _layer_call(p, b2.reshape(1, D), W3)
    p3 = _agg3_kernel(hs, row_all, col_all, norm3)

    pr = jnp.stack([p3[:, 0, ACE_IDX % 8, :], p3[:, 1, H2_IDX % 8, :]],
                   axis=1)
    wag = jnp.concatenate([Wa, Wg], axis=1)
    bag = jnp.stack([ba[0], bg[0]]).reshape(1, 2)
    out22 = _head_call(pr, b3.reshape(1, D), Wh, bh.reshape(1, D), wag, bag)
    return jnp.stack([out22[0, 0], out22[1, 1]])
